# Initial kernel scaffold; baseline (speedup 1.0000x reference)
import jax
import jax.numpy as jnp
from jax.experimental import pallas as pl


def kernel(node_embedding, W1, b1, W2, b2, W3, b3, node_type, num_variable):
    raise NotImplementedError("write your pallas kernel here")


# speedup vs baseline: 32.7793x; 32.7793x over previous
import functools

import jax
import jax.numpy as jnp
from jax import lax
from jax.experimental import pallas as pl
from jax.experimental.pallas import tpu as pltpu
from jax.experimental.pallas import tpu_sc as plsc

_NC = 2
_NS = 16
_NW = _NC * _NS
_LANES = 16
_CH = 128


def _make_seg_sum(N, H, B):
    segs_per_w = B // _NW
    lanes_per_row = H // _LANES
    mesh = plsc.VectorSubcoreMesh(core_axis_name="c", subcore_axis_name="s")

    @functools.partial(
        pl.kernel,
        mesh=mesh,
        out_type=jax.ShapeDtypeStruct((B * H,), jnp.float32),
        scratch_types=[
            pltpu.VMEM((16,), jnp.int32),
            pltpu.VMEM((_CH * H,), jnp.float32),
            pltpu.VMEM((segs_per_w * H,), jnp.float32),
        ],
    )
    def seg_sum(emb_hbm, tbl_hbm, out_hbm, tblv, buf, outv):
        wid = lax.axis_index("s") * _NC + lax.axis_index("c")
        pltpu.sync_copy(tbl_hbm.at[pl.ds(wid * 16, 16)], tblv)
        vec = tblv[...]
        bounds = [vec[j] for j in range(segs_per_w + 1)]

        for j in range(segs_per_w):
            s = bounds[j]
            e = bounds[j + 1]
            nch = (e - s + _CH - 1) // _CH

            def chunk_body(k, accs, s=s, e=e):
                lo = s + k * _CH
                o = jnp.minimum(lo, N - _CH)
                pltpu.sync_copy(emb_hbm.at[pl.ds(o * H, _CH * H)], buf)

                def row_body(r, accs):
                    g = o + r
                    valid = (g >= lo) & (g < e)
                    mv = jnp.full((16,), jnp.where(valid, 1.0, 0.0),
                                  dtype=jnp.float32)
                    base = r * H
                    return tuple(
                        accs[l] + buf[pl.ds(base + l * 16, 16)] * mv
                        for l in range(lanes_per_row)
                    )

                return lax.fori_loop(0, _CH, row_body, accs)

            accs = lax.fori_loop(
                0, nch, chunk_body,
                tuple(jnp.zeros((16,), jnp.float32)
                      for _ in range(lanes_per_row)))
            for l in range(lanes_per_row):
                outv[pl.ds(j * H + l * 16, 16)] = accs[l]

        pltpu.sync_copy(
            outv, out_hbm.at[pl.ds(wid * segs_per_w * H, segs_per_w * H)])

    return seg_sum


def _mlp_body(s_ref, c_ref, w1_ref, b1_ref, w2_ref, b2_ref, w3_ref, b3_ref,
              o_ref):
    cnt = jnp.maximum(c_ref[...], 1.0)
    x = s_ref[...] / cnt
    h = jnp.dot(x, w1_ref[...], preferred_element_type=jnp.float32)
    h = jnp.maximum(h + b1_ref[...], 0.0)
    h = jnp.dot(h, w2_ref[...], preferred_element_type=jnp.float32)
    h = jnp.maximum(h + b2_ref[...], 0.0)
    o = jnp.sum(h * w3_ref[...], axis=1, keepdims=True) + b3_ref[...]
    o_ref[...] = 1.0 / (1.0 + jnp.exp(-o))


def kernel(node_embedding, W1, b1, W2, b2, W3, b3, node_type, num_variable):
    N, H = node_embedding.shape
    B = num_variable.shape[0]
    segs_per_w = B // _NW

    offsets = jnp.concatenate(
        [jnp.zeros((1,), jnp.int32), jnp.cumsum(num_variable, dtype=jnp.int32)])
    idx = segs_per_w * jnp.arange(_NW)[:, None] + jnp.arange(segs_per_w + 1)
    tbl = jnp.pad(offsets[idx], ((0, 0), (0, 16 - (segs_per_w + 1))))

    sums = _make_seg_sum(N, H, B)(
        node_embedding.reshape(-1), tbl.reshape(-1).astype(jnp.int32))
    sums = sums.reshape(B, H)

    out = pl.pallas_call(
        _mlp_body,
        out_shape=jax.ShapeDtypeStruct((B, 1), jnp.float32),
    )(
        sums,
        num_variable.astype(jnp.float32).reshape(B, 1),
        W1, b1.reshape(1, H),
        W2, b2.reshape(1, H),
        W3.reshape(1, H),
        b3.reshape(1, 1),
    )
    return out.reshape(B)


# speedup vs baseline: 63.1330x; 1.9260x over previous
import functools

import jax
import jax.numpy as jnp
from jax import lax
from jax.experimental import pallas as pl
from jax.experimental.pallas import tpu as pltpu
from jax.experimental.pallas import tpu_sc as plsc

_NC = 2
_NS = 16
_NW = _NC * _NS
_LANES = 16
_CH = 128


def _make_seg_sum(N, H, B):
    segs_per_w = B // _NW
    lanes_per_row = H // _LANES
    mesh = plsc.VectorSubcoreMesh(core_axis_name="c", subcore_axis_name="s")

    @functools.partial(
        pl.kernel,
        mesh=mesh,
        out_type=jax.ShapeDtypeStruct((B * H,), jnp.float32),
        scratch_types=[
            pltpu.VMEM((16,), jnp.int32),
            pltpu.VMEM((_CH, H), jnp.float32),
            pltpu.VMEM((segs_per_w * H,), jnp.float32),
        ],
    )
    def seg_sum(emb_hbm, tbl_hbm, out_hbm, tblv, buf, outv):
        wid = lax.axis_index("s") * _NC + lax.axis_index("c")
        pltpu.sync_copy(tbl_hbm.at[pl.ds(wid * 16, 16)], tblv)
        vec = tblv[...]
        bounds = [vec[j] for j in range(segs_per_w + 1)]

        for j in range(segs_per_w):
            s = bounds[j]
            e = bounds[j + 1]
            a = (s // 8) * 8
            nch = (e - a + _CH - 1) // _CH

            def chunk_body(k, accs, s=s, e=e, a=a):
                lo = a + k * _CH
                o = pl.multiple_of(jnp.minimum(lo, N - _CH), 8)
                pltpu.sync_copy(emb_hbm.at[pl.ds(o, _CH)], buf)

                def row_body(r, accs):
                    g = o + r
                    valid = (g >= lo) & (g >= s) & (g < e)
                    mv = jnp.full((16,), jnp.where(valid, 1.0, 0.0),
                                  dtype=jnp.float32)
                    return tuple(
                        accs[l] + buf[r, pl.ds(l * 16, 16)] * mv
                        for l in range(lanes_per_row)
                    )

                return lax.fori_loop(0, _CH, row_body, accs)

            accs = lax.fori_loop(
                0, nch, chunk_body,
                tuple(jnp.zeros((16,), jnp.float32)
                      for _ in range(lanes_per_row)))
            for l in range(lanes_per_row):
                outv[pl.ds(j * H + l * 16, 16)] = accs[l]

        pltpu.sync_copy(
            outv, out_hbm.at[pl.ds(wid * segs_per_w * H, segs_per_w * H)])

    return seg_sum


def _mlp_body(s_ref, c_ref, w1_ref, b1_ref, w2_ref, b2_ref, w3_ref, b3_ref,
              o_ref):
    cnt = jnp.maximum(c_ref[...], 1.0)
    x = s_ref[...] / cnt
    h = jnp.dot(x, w1_ref[...], preferred_element_type=jnp.float32)
    h = jnp.maximum(h + b1_ref[...], 0.0)
    h = jnp.dot(h, w2_ref[...], preferred_element_type=jnp.float32)
    h = jnp.maximum(h + b2_ref[...], 0.0)
    o = jnp.sum(h * w3_ref[...], axis=1, keepdims=True) + b3_ref[...]
    o_ref[...] = 1.0 / (1.0 + jnp.exp(-o))


def kernel(node_embedding, W1, b1, W2, b2, W3, b3, node_type, num_variable):
    N, H = node_embedding.shape
    B = num_variable.shape[0]
    segs_per_w = B // _NW

    offsets = jnp.concatenate(
        [jnp.zeros((1,), jnp.int32), jnp.cumsum(num_variable, dtype=jnp.int32)])
    idx = segs_per_w * jnp.arange(_NW)[:, None] + jnp.arange(segs_per_w + 1)
    tbl = jnp.pad(offsets[idx], ((0, 0), (0, 16 - (segs_per_w + 1))))

    sums = _make_seg_sum(N, H, B)(
        node_embedding, tbl.reshape(-1).astype(jnp.int32))
    sums = sums.reshape(B, H)

    out = pl.pallas_call(
        _mlp_body,
        out_shape=jax.ShapeDtypeStruct((B, 1), jnp.float32),
    )(
        sums,
        num_variable.astype(jnp.float32).reshape(B, 1),
        W1, b1.reshape(1, H),
        W2, b2.reshape(1, H),
        W3.reshape(1, H),
        b3.reshape(1, 1),
    )
    return out.reshape(B)
